# baseline (speedup 1.0000x reference)
import functools

import jax
import jax.numpy as jnp
from jax import lax
from jax.experimental import pallas as pl
from jax.experimental.pallas import tpu as pltpu
from jax.experimental.pallas import tpu_sc as plsc

NC, NS = 2, 16
NTILES = NC * NS
CH = 128
BE = 1024
BN = 632

F32 = jnp.float32


def _silu(x):
    return x * jax.nn.sigmoid(x)


def _mesh():
    return plsc.VectorSubcoreMesh(
        core_axis_name="c", subcore_axis_name="s", num_cores=NC, num_subcores=NS
    )


@functools.lru_cache(maxsize=None)
def _make_gather(n1, ep):
    per_tile = ep // NTILES
    nch = per_tile // CH
    assert nch % 2 == 0

    @functools.partial(
        pl.kernel,
        out_type=[
            jax.ShapeDtypeStruct((ep, 144), F32),
            jax.ShapeDtypeStruct((ep, 144), F32),
        ],
        mesh=_mesh(),
        scratch_types=(
            [pltpu.VMEM((CH,), jnp.int32)] * 4
            + [pltpu.VMEM((CH, 144), F32)] * 4
            + [pltpu.SemaphoreType.DMA] * 6
        ),
        compiler_params=pltpu.CompilerParams(use_tc_tiling_on_sc=False),
    )
    def gather_k(Tc, Tr, colp, rowp, gA, gB,
                 colv0, colv1, rowv0, rowv1, bA0, bA1, bB0, bB1,
                 is0, is1, gs0, gs1, ws0, ws1):
        wid = lax.axis_index("s") * NC + lax.axis_index("c")
        base0 = wid * per_tile
        bufs = [
            dict(colv=colv0, rowv=rowv0, A=bA0, B=bB0,
                 isem=is0, gsem=gs0, wsem=ws0),
            dict(colv=colv1, rowv=rowv1, A=bA1, B=bB1,
                 isem=is1, gsem=gs1, wsem=ws1),
        ]

        def load_idx(k, b):
            base = base0 + k * CH
            pltpu.async_copy(colp.at[pl.ds(base, CH)], b["colv"], b["isem"])
            pltpu.async_copy(rowp.at[pl.ds(base, CH)], b["rowv"], b["isem"])

        def drain_idx(b):
            pltpu.make_async_copy(colp.at[pl.ds(0, CH)], b["colv"], b["isem"]).wait()
            pltpu.make_async_copy(rowp.at[pl.ds(0, CH)], b["rowv"], b["isem"]).wait()

        def fire_g(b):
            pltpu.async_copy(Tc.at[b["colv"]], b["A"], b["gsem"])
            pltpu.async_copy(Tr.at[b["rowv"]], b["B"], b["gsem"])

        def drain_g(b):
            pltpu.make_async_copy(Tc.at[b["colv"]], b["A"], b["gsem"]).wait()
            pltpu.make_async_copy(Tr.at[b["rowv"]], b["B"], b["gsem"]).wait()

        def fire_wb(k, b):
            base = base0 + k * CH
            pltpu.async_copy(b["A"], gA.at[pl.ds(base, CH)], b["wsem"])
            pltpu.async_copy(b["B"], gB.at[pl.ds(base, CH)], b["wsem"])

        def drain_wb(b):
            pltpu.make_async_copy(b["A"], gA.at[pl.ds(0, CH)], b["wsem"]).wait()
            pltpu.make_async_copy(b["B"], gB.at[pl.ds(0, CH)], b["wsem"]).wait()

        load_idx(0, bufs[0])
        drain_idx(bufs[0])
        fire_g(bufs[0])
        load_idx(1, bufs[1])

        def pair(p, carry):
            for i in (0, 1):
                k = 2 * p + i
                b, bo = bufs[i], bufs[1 - i]
                drain_g(b)

                @pl.when(k + 1 < nch)
                def _start_next():
                    @pl.when(k >= 1)
                    def _():
                        drain_wb(bo)
                    drain_idx(bo)
                    fire_g(bo)

                fire_wb(k, b)

                @pl.when(k + 2 < nch)
                def _prefetch():
                    load_idx(k + 2, b)

            return carry

        lax.fori_loop(0, nch // 2, pair, 0)
        drain_wb(bufs[0])
        drain_wb(bufs[1])

    return gather_k


@functools.lru_cache(maxsize=None)
def _make_scatter(n1, ep):
    per_core = ep // NC
    per_tile = per_core // NS
    nch = per_tile // CH
    rows_pt = n1 // NS

    @functools.partial(
        pl.kernel,
        out_type=[
            jax.ShapeDtypeStruct((NC * n1, 128), F32),
            jax.ShapeDtypeStruct((NC * n1, 16), F32),
        ],
        mesh=_mesh(),
        scratch_types=(
            [pltpu.VMEM((CH,), jnp.int32)] * 4
            + [pltpu.VMEM((CH, 128), F32)] * 2
            + [pltpu.VMEM((CH, 16), F32)] * 2
            + [
                pltpu.VMEM_SHARED((n1, 128), F32),
                pltpu.VMEM_SHARED((n1, 16), F32),
            ]
            + [pltpu.SemaphoreType.DMA] * 4
        ),
        compiler_params=pltpu.CompilerParams(use_tc_tiling_on_sc=False),
    )
    def scatter_k(m, trans, colp, rowp, z128, z16, aggr_out, delta_out,
                  colv0, colv1, rowv0, rowv1, bM0, bM1, bT0, bT1,
                  shA, shD, ls0, ls1, ss0, ss1):
        cid = lax.axis_index("c")
        sid = lax.axis_index("s")

        @pl.when(sid == 0)
        def _init():
            pltpu.sync_copy(z128, shA)
            pltpu.sync_copy(z16, shD)

        plsc.subcore_barrier()

        base0 = cid * per_core + sid * per_tile
        bufs = [
            dict(colv=colv0, rowv=rowv0, M=bM0, T=bT0, lsem=ls0, ssem=ss0),
            dict(colv=colv1, rowv=rowv1, M=bM1, T=bT1, lsem=ls1, ssem=ss1),
        ]

        def load(k, b):
            base = base0 + k * CH
            pltpu.async_copy(colp.at[pl.ds(base, CH)], b["colv"], b["lsem"])
            pltpu.async_copy(rowp.at[pl.ds(base, CH)], b["rowv"], b["lsem"])
            pltpu.async_copy(m.at[pl.ds(base, CH)], b["M"], b["lsem"])
            pltpu.async_copy(trans.at[pl.ds(base, CH)], b["T"], b["lsem"])

        def drain_load(b):
            pltpu.make_async_copy(colp.at[pl.ds(0, CH)], b["colv"], b["lsem"]).wait()
            pltpu.make_async_copy(rowp.at[pl.ds(0, CH)], b["rowv"], b["lsem"]).wait()
            pltpu.make_async_copy(m.at[pl.ds(0, CH)], b["M"], b["lsem"]).wait()
            pltpu.make_async_copy(trans.at[pl.ds(0, CH)], b["T"], b["lsem"]).wait()

        def fire_scat(b):
            pltpu.async_copy(b["M"], shA.at[b["colv"]], b["ssem"], add=True)
            pltpu.async_copy(b["T"], shD.at[b["rowv"]], b["ssem"], add=True)

        def drain_scat(b):
            pltpu.make_async_copy(b["M"], shA.at[b["colv"]], b["ssem"]).wait()
            pltpu.make_async_copy(b["T"], shD.at[b["rowv"]], b["ssem"]).wait()

        load(0, bufs[0])

        def pair(p, carry):
            for i in (0, 1):
                k = 2 * p + i
                b, bo = bufs[i], bufs[1 - i]
                drain_load(b)

                @pl.when(k + 1 < nch)
                def _start_next():
                    @pl.when(k >= 1)
                    def _():
                        drain_scat(bo)
                    load(k + 1, bo)

                fire_scat(b)
            return carry

        lax.fori_loop(0, nch // 2, pair, 0)
        drain_scat(bufs[0])
        drain_scat(bufs[1])
        plsc.subcore_barrier()

        r0 = sid * rows_pt
        pltpu.sync_copy(shA.at[pl.ds(r0, rows_pt)],
                        aggr_out.at[pl.ds(cid * n1 + r0, rows_pt)])
        pltpu.sync_copy(shD.at[pl.ds(r0, rows_pt)],
                        delta_out.at[pl.ds(cid * n1 + r0, rows_pt)])

    return scatter_k


def _mm(a, b):
    return jnp.dot(a, b, preferred_element_type=F32)


def _geometry_packed(xrP, xcP, P1B, P2B, OmB):
    dx = xrP - xcP
    norm = jnp.sqrt(_mm(dx * dx, OmB) + 1e-8) + 1.0
    cd = dx / norm
    ccr = _mm(xrP, P1B) * _mm(xcP, P2B) - _mm(xrP, P2B) * _mm(xcP, P1B)
    cn = jnp.sqrt(_mm(ccr * ccr, OmB) + 1e-8) + 1.0
    cc = ccr / cn
    cv = _mm(cd, P1B) * _mm(cc, P2B) - _mm(cd, P2B) * _mm(cc, P1B)
    return cd, cc, cv


def _full_spec(shape):
    nd = len(shape)
    return pl.BlockSpec(shape, lambda i, _nd=nd: (0,) * _nd)


def _row_spec(block_rows, cols):
    return pl.BlockSpec((block_rows, cols), lambda i: (i, 0))


@functools.lru_cache(maxsize=None)
def _make_ef0(ep):
    grid = ep // BE
    bp = BE // 8

    def body(xcgP, xrgP, P1B, P2B, OmB, MfP, bf1P, Wf2P, bf2P, ef_ref):
        xr = xrgP[...]
        xc = xcgP[...]
        p1b, p2b, omb = P1B[...], P2B[...], OmB[...]
        cd, cc, cv = _geometry_packed(xr, xc, p1b, p2b, omb)
        ci0 = _mm(cd * xr, omb)
        ci1 = _mm(cc * xr, omb)
        ci2 = _mm(cv * xr, omb)
        cj0 = _mm(cd * xc, omb)
        cj1 = _mm(cc * xc, omb)
        cj2 = _mm(cv * xc, omb)
        ni = jnp.sqrt(ci0 * ci0 + ci1 * ci1 + ci2 * ci2) + 1e-5
        nj = jnp.sqrt(cj0 * cj0 + cj1 * cj1 + cj2 * cj2) + 1e-5
        pcos = (ci0 * cj0 + ci1 * cj1 + ci2 * cj2) / (ni * nj)
        psin = jnp.sqrt(jnp.clip(1.0 - pcos * pcos, 0.0, None))
        v = jnp.concatenate(
            [psin, pcos, ci0, ci1, ci2, cj0, cj1, cj2], axis=1)
        ef = _silu(_mm(v, MfP[...]) + bf1P[...])
        ef_ref[...] = _silu(_mm(ef, Wf2P[...]) + bf2P[...])

    return pl.pallas_call(
        body,
        grid=(grid,),
        in_specs=[
            _row_spec(bp, 128), _row_spec(bp, 128),
            _full_spec((128, 128)), _full_spec((128, 128)),
            _full_spec((128, 128)),
            _full_spec((1024, 512)), _full_spec((1, 512)),
            _full_spec((512, 512)), _full_spec((1, 512)),
        ],
        out_specs=[_row_spec(bp, 512)],
        out_shape=[jax.ShapeDtypeStruct((ep // 8, 512), F32)],
    )


@functools.lru_cache(maxsize=None)
def _make_edge(ep):
    grid = ep // BE

    def body(gA, gB, ef, Or, w1c, W1d, b1, W2, b2, Wc1, bc1,
             Wc2, bc2, m_ref, co_ref):
        a = gA[...]
        b = gB[...]
        dx16 = b[:, 128:144] - a[:, 128:144]
        radial128 = _mm(dx16 * dx16, Or[...])
        pre = (a[:, 0:128] + b[:, 0:128] + radial128 * w1c[...]
               + _mm(ef[...], W1d[...]) + b1[...])
        m1 = _silu(pre)
        m = _silu(_mm(m1, W2[...]) + b2[...])
        c1h = _silu(_mm(m, Wc1[...]) + bc1[...])
        co_ref[...] = _mm(c1h, Wc2[...]) + bc2[...]
        m_ref[...] = m

    return pl.pallas_call(
        body,
        grid=(grid,),
        in_specs=[
            _row_spec(BE, 144), _row_spec(BE, 144),
            _row_spec(BE, 64),
            _full_spec((16, 128)),
            _full_spec((1, 128)), _full_spec((64, 128)), _full_spec((1, 128)),
            _full_spec((128, 128)), _full_spec((1, 128)),
            _full_spec((128, 128)), _full_spec((1, 128)),
            _full_spec((128, 16)), _full_spec((1, 16)),
        ],
        out_specs=[_row_spec(BE, 128), _row_spec(BE, 16)],
        out_shape=[
            jax.ShapeDtypeStruct((ep, 128), F32),
            jax.ShapeDtypeStruct((ep, 16), F32),
        ],
    )


@functools.lru_cache(maxsize=None)
def _make_trans(ep):
    bp = BE // 8
    grid = ep // BE

    def body(xcgP, xrgP, coffP, P1B, P2B, OmB, SB0, SB1, SB2, t_ref):
        cd, cc, cv = _geometry_packed(
            xrgP[...], xcgP[...], P1B[...], P2B[...], OmB[...])
        cP = coffP[...]
        k0 = _mm(cP, SB0[...])
        k1 = _mm(cP, SB1[...])
        k2 = _mm(cP, SB2[...])
        t_ref[...] = cd * k0 + cc * k1 + cv * k2

    return pl.pallas_call(
        body,
        grid=(grid,),
        in_specs=[
            _row_spec(bp, 128), _row_spec(bp, 128), _row_spec(bp, 128),
            _full_spec((128, 128)), _full_spec((128, 128)),
            _full_spec((128, 128)),
            _full_spec((128, 128)), _full_spec((128, 128)),
            _full_spec((128, 128)),
        ],
        out_specs=[_row_spec(bp, 128)],
        out_shape=[jax.ShapeDtypeStruct((ep // 8, 128), F32)],
    )


@functools.lru_cache(maxsize=None)
def _make_centroid(rows, cols):
    def body(xt, xc_ref, cent_ref):
        v = xt[...]
        c = jnp.mean(v, axis=1, keepdims=True)
        cb = jnp.broadcast_to(c, v.shape)
        xc_ref[...] = v - cb
        cent_ref[...] = cb

    return pl.pallas_call(
        body,
        grid=(1,),
        in_specs=[_full_spec((rows, cols))],
        out_specs=[_full_spec((rows, cols)), _full_spec((rows, cols))],
        out_shape=[
            jax.ShapeDtypeStruct((rows, cols), F32),
            jax.ShapeDtypeStruct((rows, cols), F32),
        ],
    )


@functools.lru_cache(maxsize=None)
def _make_embed(n1):
    grid = n1 // BN

    def body(h, We, be, W1a, W1b, h1_ref, hA_ref, hB_ref):
        h1 = jnp.dot(h[...], We[...], preferred_element_type=F32) + be[...]
        h1_ref[...] = h1
        hA_ref[...] = _mm(h1, W1a[...])
        hB_ref[...] = _mm(h1, W1b[...])

    return pl.pallas_call(
        body,
        grid=(grid,),
        in_specs=[
            _row_spec(BN, 128),
            _full_spec((128, 128)), _full_spec((1, 128)),
            _full_spec((128, 128)), _full_spec((128, 128)),
        ],
        out_specs=[_row_spec(BN, 128)] * 3,
        out_shape=[
            jax.ShapeDtypeStruct((n1, 128), F32),
            jax.ShapeDtypeStruct((n1, 128), F32),
            jax.ShapeDtypeStruct((n1, 128), F32),
        ],
    )


def _node_update(h, a0, a1, Wn1a, Wn1b, bn1, Wn2, bn2, g, b):
    aggr = a0 + a1
    pre = (jnp.dot(h, Wn1a, preferred_element_type=F32)
           + jnp.dot(aggr, Wn1b, preferred_element_type=F32) + bn1)
    hn = jnp.dot(_silu(pre), Wn2, preferred_element_type=F32) + bn2
    mu = jnp.mean(hn, axis=1, keepdims=True)
    dev = hn - mu
    var = jnp.mean(dev * dev, axis=1, keepdims=True)
    return dev * jax.lax.rsqrt(var + 1e-5) * g + b


@functools.lru_cache(maxsize=None)
def _make_node(n1):
    grid = n1 // BN

    def body(h, a0, a1, d0, d1, xc, Wn1a, Wn1b, bn1, Wn2, bn2, g, b,
             W1a, W1b, h_ref, xc_ref, hA_ref, hB_ref):
        hnew = _node_update(h[...], a0[...], a1[...], Wn1a[...], Wn1b[...],
                            bn1[...], Wn2[...], bn2[...], g[...], b[...])
        h_ref[...] = hnew
        xc_ref[...] = xc[...] + d0[...] + d1[...]
        hA_ref[...] = _mm(hnew, W1a[...])
        hB_ref[...] = _mm(hnew, W1b[...])

    return pl.pallas_call(
        body,
        grid=(grid,),
        in_specs=[
            _row_spec(BN, 128),
            _row_spec(BN, 128), _row_spec(BN, 128),
            _row_spec(BN, 16), _row_spec(BN, 16), _row_spec(BN, 16),
            _full_spec((128, 128)), _full_spec((128, 128)), _full_spec((1, 128)),
            _full_spec((128, 128)), _full_spec((1, 128)),
            _full_spec((1, 128)), _full_spec((1, 128)),
            _full_spec((128, 128)), _full_spec((128, 128)),
        ],
        out_specs=[_row_spec(BN, 128), _row_spec(BN, 16),
                   _row_spec(BN, 128), _row_spec(BN, 128)],
        out_shape=[
            jax.ShapeDtypeStruct((n1, 128), F32),
            jax.ShapeDtypeStruct((n1, 16), F32),
            jax.ShapeDtypeStruct((n1, 128), F32),
            jax.ShapeDtypeStruct((n1, 128), F32),
        ],
    )


@functools.lru_cache(maxsize=None)
def _make_node_final(n1):
    grid = n1 // BN

    def body(h, a0, a1, d0, d1, xc, cent, Wn1a, Wn1b, bn1, Wn2, bn2, g, b,
             Wo, bo, ho_ref, xo_ref):
        hnew = _node_update(h[...], a0[...], a1[...], Wn1a[...], Wn1b[...],
                            bn1[...], Wn2[...], bn2[...], g[...], b[...])
        ho_ref[...] = jnp.dot(hnew, Wo[...], preferred_element_type=F32) + bo[...]
        xo_ref[...] = xc[...] + d0[...] + d1[...] + cent[...]

    return pl.pallas_call(
        body,
        grid=(grid,),
        in_specs=[
            _row_spec(BN, 128),
            _row_spec(BN, 128), _row_spec(BN, 128),
            _row_spec(BN, 16), _row_spec(BN, 16), _row_spec(BN, 16),
            _row_spec(BN, 16),
            _full_spec((128, 128)), _full_spec((128, 128)), _full_spec((1, 128)),
            _full_spec((128, 128)), _full_spec((1, 128)),
            _full_spec((1, 128)), _full_spec((1, 128)),
            _full_spec((128, 128)), _full_spec((1, 128)),
        ],
        out_specs=[_row_spec(BN, 128), _row_spec(BN, 16)],
        out_shape=[
            jax.ShapeDtypeStruct((n1, 128), F32),
            jax.ShapeDtypeStruct((n1, 16), F32),
        ],
    )


def _r2(v):
    return v.reshape(1, -1)


def kernel(h, edge_index, x, n_nodes, params):
    n, dmod = h.shape
    e = edge_index.shape[1]
    nn = 100
    nmol = n // nn

    n1 = ((n + 1 + NS * 8 - 1) // (NS * 8)) * (NS * 8)
    epm = NTILES * CH * 2
    ep = ((e + epm - 1) // epm) * epm

    row = edge_index[0].astype(jnp.int32)
    col = edge_index[1].astype(jnp.int32)
    pad_e = ep - e
    rowp = jnp.concatenate([row, jnp.full((pad_e,), n, jnp.int32)])
    colp = jnp.concatenate([col, jnp.full((pad_e,), n, jnp.int32)])

    xt = x.reshape(nmol, nn, 3).transpose(0, 2, 1).reshape(3 * nmol, nn)
    xc_rows, cent_rows = _make_centroid(3 * nmol, nn)(xt)
    xc0 = xc_rows.reshape(nmol, 3, nn).transpose(0, 2, 1).reshape(n, 3)
    cent = cent_rows.reshape(nmol, 3, nn).transpose(0, 2, 1).reshape(n, 3)
    zpadn = jnp.zeros((n1 - n, 3), F32)
    xcp = jnp.tile(jnp.concatenate([xc0, zpadn], 0), (1, 6))[:, :16]
    centp = jnp.pad(jnp.concatenate([cent, zpadn], 0), ((0, 0), (0, 13)))

    eye16 = jnp.eye(16, dtype=F32)
    eye8 = jnp.eye(8, dtype=F32)
    P1 = jnp.roll(eye16, 1, axis=0)
    P2 = jnp.roll(eye16, 2, axis=0)
    Om = jnp.concatenate([jnp.ones((3, 16), F32), jnp.zeros((13, 16), F32)], 0)
    Or = jnp.concatenate([jnp.ones((3, 128), F32), jnp.zeros((13, 128), F32)], 0)
    P1B = jnp.kron(eye8, P1)
    P2B = jnp.kron(eye8, P2)
    OmB = jnp.kron(eye8, Om)
    sels = [jnp.zeros((16, 16), F32).at[k].set(1.0) for k in range(3)]
    SB0, SB1, SB2 = (jnp.kron(eye8, s) for s in sels)
    Wf1 = params["fuse1"]["w"]
    MfP = jnp.concatenate(
        [jnp.kron(eye8, jnp.ones((16, 1), F32) @ (Wf1[j:j + 1] / 16.0))
         for j in range(8)], 0)
    Wf2P = jnp.kron(eye8, params["fuse2"]["w"])
    bf1P = jnp.tile(_r2(params["fuse1"]["b"]), (1, 8))
    bf2P = jnp.tile(_r2(params["fuse2"]["b"]), (1, 8))

    hpad = jnp.concatenate([h, jnp.zeros((n1 - n, dmod), F32)], 0)

    layers = params["layers"]

    def esplit(lp):
        W1 = lp["edge1"]["w"]
        return (_r2(W1[256]), W1[257:321],
                _r2(lp["edge1"]["b"]), lp["edge2"]["w"], _r2(lp["edge2"]["b"]),
                lp["coord1"]["w"], _r2(lp["coord1"]["b"]),
                jnp.pad(lp["coord2"]["w"], ((0, 0), (0, 13))),
                jnp.pad(_r2(lp["coord2"]["b"]), ((0, 0), (0, 13))))

    def nsplit(lp):
        Wn1 = lp["node1"]["w"]
        return (Wn1[0:128], Wn1[128:256], _r2(lp["node1"]["b"]),
                lp["node2"]["w"], _r2(lp["node2"]["b"]),
                _r2(lp["ln_g"]), _r2(lp["ln_b"]))

    W1a0, W1b0 = layers[0]["edge1"]["w"][0:128], layers[0]["edge1"]["w"][128:256]
    hcur, hA, hB = _make_embed(n1)(
        hpad, params["emb_in"]["w"], _r2(params["emb_in"]["b"]), W1a0, W1b0)

    zeros128 = jnp.zeros((n1, 128), F32)
    zeros16 = jnp.zeros((n1, 16), F32)

    gather = _make_gather(n1, ep)
    scatter = _make_scatter(n1, ep)
    ef0 = _make_ef0(ep)
    edgeK = _make_edge(ep)
    transK = _make_trans(ep)
    nodeK = _make_node(n1)
    nodeF = _make_node_final(n1)

    ef = None
    for li in range(4):
        lp = layers[li]
        gA, gB = gather(jnp.concatenate([hA, xcp], 1),
                        jnp.concatenate([hB, xcp], 1), colp, rowp)
        xcgP = gA[:, 128:144].reshape(ep // 8, 128)
        xrgP = gB[:, 128:144].reshape(ep // 8, 128)
        ew = esplit(lp)
        if li == 0:
            (efP,) = ef0(xcgP, xrgP, P1B, P2B, OmB, MfP, bf1P, Wf2P, bf2P)
            ef = efP.reshape(ep, 64)
        m, coff = edgeK(gA, gB, ef, Or, *ew)
        coffP = coff.reshape(ep // 8, 128)
        (transP,) = transK(xcgP, xrgP, coffP, P1B, P2B, OmB, SB0, SB1, SB2)
        trans = transP.reshape(ep, 16)
        aggr2, delta2 = scatter(m, trans, colp, rowp, zeros128, zeros16)
        a0, a1 = aggr2[:n1], aggr2[n1:]
        d0, d1 = delta2[:n1], delta2[n1:]
        nw = nsplit(lp)
        if li < 3:
            Wn = layers[li + 1]["edge1"]["w"]
            hcur, xcp, hA, hB = nodeK(
                hcur, a0, a1, d0, d1, xcp, *nw, Wn[0:128], Wn[128:256])
        else:
            ho, xo16 = nodeF(
                hcur, a0, a1, d0, d1, xcp, centp, *nw,
                params["emb_out"]["w"], _r2(params["emb_out"]["b"]))

    return ho[:n], xo16[:n, :3]


# speedup vs baseline: 1.4005x; 1.4005x over previous
import functools

import jax
import jax.numpy as jnp
from jax import lax
from jax.experimental import pallas as pl
from jax.experimental.pallas import tpu as pltpu
from jax.experimental.pallas import tpu_sc as plsc

NC, NS = 2, 16
NTILES = NC * NS
CH = 128
BE = 1024
BN = 632

F32 = jnp.float32


def _silu(x):
    return x * jax.nn.sigmoid(x)


def _mesh():
    return plsc.VectorSubcoreMesh(
        core_axis_name="c", subcore_axis_name="s", num_cores=NC, num_subcores=NS
    )


@functools.lru_cache(maxsize=None)
def _make_gather(n1, ep):
    per_tile = ep // NTILES
    nch = per_tile // CH
    assert nch % 2 == 0

    @functools.partial(
        pl.kernel,
        out_type=[
            jax.ShapeDtypeStruct((ep, 128), F32),
            jax.ShapeDtypeStruct((ep, 128), F32),
            jax.ShapeDtypeStruct((ep, 16), F32),
            jax.ShapeDtypeStruct((ep, 16), F32),
        ],
        mesh=_mesh(),
        scratch_types=(
            [pltpu.VMEM((CH,), jnp.int32)] * 4
            + [pltpu.VMEM((CH, 128), F32)] * 4
            + [pltpu.VMEM((CH, 16), F32)] * 4
            + [pltpu.SemaphoreType.DMA] * 6
        ),
        compiler_params=pltpu.CompilerParams(use_tc_tiling_on_sc=False),
    )
    def gather_k(hA, hB, xcp, colp, rowp, sA, sB, xcg, xrg,
                 colv0, colv1, rowv0, rowv1, bA0, bA1, bB0, bB1,
                 bC0, bC1, bR0, bR1, is0, is1, gs0, gs1, ws0, ws1):
        wid = lax.axis_index("s") * NC + lax.axis_index("c")
        base0 = wid * per_tile
        bufs = [
            dict(colv=colv0, rowv=rowv0, A=bA0, B=bB0, C=bC0, R=bR0,
                 isem=is0, gsem=gs0, wsem=ws0),
            dict(colv=colv1, rowv=rowv1, A=bA1, B=bB1, C=bC1, R=bR1,
                 isem=is1, gsem=gs1, wsem=ws1),
        ]

        def load_idx(k, b):
            base = base0 + k * CH
            pltpu.async_copy(colp.at[pl.ds(base, CH)], b["colv"], b["isem"])
            pltpu.async_copy(rowp.at[pl.ds(base, CH)], b["rowv"], b["isem"])

        def drain_idx(b):
            pltpu.make_async_copy(colp.at[pl.ds(0, CH)], b["colv"], b["isem"]).wait()
            pltpu.make_async_copy(rowp.at[pl.ds(0, CH)], b["rowv"], b["isem"]).wait()

        def fire_g(b):
            pltpu.async_copy(hA.at[b["colv"]], b["A"], b["gsem"])
            pltpu.async_copy(hB.at[b["rowv"]], b["B"], b["gsem"])
            pltpu.async_copy(xcp.at[b["colv"]], b["C"], b["gsem"])
            pltpu.async_copy(xcp.at[b["rowv"]], b["R"], b["gsem"])

        def drain_g(b):
            pltpu.make_async_copy(hA.at[b["colv"]], b["A"], b["gsem"]).wait()
            pltpu.make_async_copy(hB.at[b["rowv"]], b["B"], b["gsem"]).wait()
            pltpu.make_async_copy(xcp.at[b["colv"]], b["C"], b["gsem"]).wait()
            pltpu.make_async_copy(xcp.at[b["rowv"]], b["R"], b["gsem"]).wait()

        def fire_wb(k, b):
            base = base0 + k * CH
            pltpu.async_copy(b["A"], sA.at[pl.ds(base, CH)], b["wsem"])
            pltpu.async_copy(b["B"], sB.at[pl.ds(base, CH)], b["wsem"])
            pltpu.async_copy(b["C"], xcg.at[pl.ds(base, CH)], b["wsem"])
            pltpu.async_copy(b["R"], xrg.at[pl.ds(base, CH)], b["wsem"])

        def drain_wb(b):
            pltpu.make_async_copy(b["A"], sA.at[pl.ds(0, CH)], b["wsem"]).wait()
            pltpu.make_async_copy(b["B"], sB.at[pl.ds(0, CH)], b["wsem"]).wait()
            pltpu.make_async_copy(b["C"], xcg.at[pl.ds(0, CH)], b["wsem"]).wait()
            pltpu.make_async_copy(b["R"], xrg.at[pl.ds(0, CH)], b["wsem"]).wait()

        load_idx(0, bufs[0])
        drain_idx(bufs[0])
        fire_g(bufs[0])
        load_idx(1, bufs[1])

        def pair(p, carry):
            for i in (0, 1):
                k = 2 * p + i
                b, bo = bufs[i], bufs[1 - i]
                drain_g(b)

                @pl.when(k + 1 < nch)
                def _start_next():
                    @pl.when(k >= 1)
                    def _():
                        drain_wb(bo)
                    drain_idx(bo)
                    fire_g(bo)

                fire_wb(k, b)

                @pl.when(k + 2 < nch)
                def _prefetch():
                    load_idx(k + 2, b)

            return carry

        lax.fori_loop(0, nch // 2, pair, 0)
        drain_wb(bufs[0])
        drain_wb(bufs[1])

    return gather_k


@functools.lru_cache(maxsize=None)
def _make_scatter(n1, ep):
    per_core = ep // NC
    per_tile = per_core // NS
    nch = per_tile // CH
    rows_pt = n1 // NS

    @functools.partial(
        pl.kernel,
        out_type=[
            jax.ShapeDtypeStruct((NC * n1, 128), F32),
            jax.ShapeDtypeStruct((NC * n1, 16), F32),
        ],
        mesh=_mesh(),
        scratch_types=(
            [pltpu.VMEM((CH,), jnp.int32)] * 4
            + [pltpu.VMEM((CH, 128), F32)] * 2
            + [pltpu.VMEM((CH, 16), F32)] * 2
            + [
                pltpu.VMEM_SHARED((n1, 128), F32),
                pltpu.VMEM_SHARED((n1, 16), F32),
            ]
            + [pltpu.SemaphoreType.DMA] * 4
        ),
        compiler_params=pltpu.CompilerParams(use_tc_tiling_on_sc=False),
    )
    def scatter_k(m, trans, colp, rowp, z128, z16, aggr_out, delta_out,
                  colv0, colv1, rowv0, rowv1, bM0, bM1, bT0, bT1,
                  shA, shD, ls0, ls1, ss0, ss1):
        cid = lax.axis_index("c")
        sid = lax.axis_index("s")

        @pl.when(sid == 0)
        def _init():
            pltpu.sync_copy(z128, shA)
            pltpu.sync_copy(z16, shD)

        plsc.subcore_barrier()

        base0 = cid * per_core + sid * per_tile
        bufs = [
            dict(colv=colv0, rowv=rowv0, M=bM0, T=bT0, lsem=ls0, ssem=ss0),
            dict(colv=colv1, rowv=rowv1, M=bM1, T=bT1, lsem=ls1, ssem=ss1),
        ]

        def load(k, b):
            base = base0 + k * CH
            pltpu.async_copy(colp.at[pl.ds(base, CH)], b["colv"], b["lsem"])
            pltpu.async_copy(rowp.at[pl.ds(base, CH)], b["rowv"], b["lsem"])
            pltpu.async_copy(m.at[pl.ds(base, CH)], b["M"], b["lsem"])
            pltpu.async_copy(trans.at[pl.ds(base, CH)], b["T"], b["lsem"])

        def drain_load(b):
            pltpu.make_async_copy(colp.at[pl.ds(0, CH)], b["colv"], b["lsem"]).wait()
            pltpu.make_async_copy(rowp.at[pl.ds(0, CH)], b["rowv"], b["lsem"]).wait()
            pltpu.make_async_copy(m.at[pl.ds(0, CH)], b["M"], b["lsem"]).wait()
            pltpu.make_async_copy(trans.at[pl.ds(0, CH)], b["T"], b["lsem"]).wait()

        def fire_scat(b):
            pltpu.async_copy(b["M"], shA.at[b["colv"]], b["ssem"], add=True)
            pltpu.async_copy(b["T"], shD.at[b["rowv"]], b["ssem"], add=True)

        def drain_scat(b):
            pltpu.make_async_copy(b["M"], shA.at[b["colv"]], b["ssem"]).wait()
            pltpu.make_async_copy(b["T"], shD.at[b["rowv"]], b["ssem"]).wait()

        load(0, bufs[0])

        def pair(p, carry):
            for i in (0, 1):
                k = 2 * p + i
                b, bo = bufs[i], bufs[1 - i]
                drain_load(b)

                @pl.when(k + 1 < nch)
                def _start_next():
                    @pl.when(k >= 1)
                    def _():
                        drain_scat(bo)
                    load(k + 1, bo)

                fire_scat(b)
            return carry

        lax.fori_loop(0, nch // 2, pair, 0)
        drain_scat(bufs[0])
        drain_scat(bufs[1])
        plsc.subcore_barrier()

        r0 = sid * rows_pt
        pltpu.sync_copy(shA.at[pl.ds(r0, rows_pt)],
                        aggr_out.at[pl.ds(cid * n1 + r0, rows_pt)])
        pltpu.sync_copy(shD.at[pl.ds(r0, rows_pt)],
                        delta_out.at[pl.ds(cid * n1 + r0, rows_pt)])

    return scatter_k


def _mm(a, b):
    return jnp.dot(a, b, preferred_element_type=F32)


def _geometry_packed(xrP, xcP, P1B, P2B, OmB):
    dx = xrP - xcP
    norm = jnp.sqrt(_mm(dx * dx, OmB) + 1e-8) + 1.0
    cd = dx / norm
    ccr = _mm(xrP, P1B) * _mm(xcP, P2B) - _mm(xrP, P2B) * _mm(xcP, P1B)
    cn = jnp.sqrt(_mm(ccr * ccr, OmB) + 1e-8) + 1.0
    cc = ccr / cn
    cv = _mm(cd, P1B) * _mm(cc, P2B) - _mm(cd, P2B) * _mm(cc, P1B)
    return cd, cc, cv


def _full_spec(shape):
    nd = len(shape)
    return pl.BlockSpec(shape, lambda i, _nd=nd: (0,) * _nd)


def _row_spec(block_rows, cols):
    return pl.BlockSpec((block_rows, cols), lambda i: (i, 0))


@functools.lru_cache(maxsize=None)
def _make_ef0(ep):
    grid = ep // BE
    bp = BE // 8

    def body(xcgP, xrgP, P1B, P2B, OmB, MfP, bf1P, Wf2P, bf2P, ef_ref):
        xr = xrgP[...]
        xc = xcgP[...]
        p1b, p2b, omb = P1B[...], P2B[...], OmB[...]
        cd, cc, cv = _geometry_packed(xr, xc, p1b, p2b, omb)
        ci0 = _mm(cd * xr, omb)
        ci1 = _mm(cc * xr, omb)
        ci2 = _mm(cv * xr, omb)
        cj0 = _mm(cd * xc, omb)
        cj1 = _mm(cc * xc, omb)
        cj2 = _mm(cv * xc, omb)
        ni = jnp.sqrt(ci0 * ci0 + ci1 * ci1 + ci2 * ci2) + 1e-5
        nj = jnp.sqrt(cj0 * cj0 + cj1 * cj1 + cj2 * cj2) + 1e-5
        pcos = (ci0 * cj0 + ci1 * cj1 + ci2 * cj2) / (ni * nj)
        psin = jnp.sqrt(jnp.clip(1.0 - pcos * pcos, 0.0, None))
        v = jnp.concatenate(
            [psin, pcos, ci0, ci1, ci2, cj0, cj1, cj2], axis=1)
        ef = _silu(_mm(v, MfP[...]) + bf1P[...])
        ef_ref[...] = _silu(_mm(ef, Wf2P[...]) + bf2P[...])

    return pl.pallas_call(
        body,
        grid=(grid,),
        in_specs=[
            _row_spec(bp, 128), _row_spec(bp, 128),
            _full_spec((128, 128)), _full_spec((128, 128)),
            _full_spec((128, 128)),
            _full_spec((1024, 512)), _full_spec((1, 512)),
            _full_spec((512, 512)), _full_spec((1, 512)),
        ],
        out_specs=[_row_spec(bp, 512)],
        out_shape=[jax.ShapeDtypeStruct((ep // 8, 512), F32)],
    )


@functools.lru_cache(maxsize=None)
def _make_edge(ep):
    grid = ep // BE

    def body(sA, sB, xcg, xrg, ef, Or, w1c, W1d, b1, W2, b2, Wc1, bc1,
             Wc2, bc2, m_ref, co_ref):
        dx16 = xrg[...] - xcg[...]
        radial128 = _mm(dx16 * dx16, Or[...])
        pre = (sA[...] + sB[...] + radial128 * w1c[...]
               + _mm(ef[...], W1d[...]) + b1[...])
        m1 = _silu(pre)
        m = _silu(_mm(m1, W2[...]) + b2[...])
        c1h = _silu(_mm(m, Wc1[...]) + bc1[...])
        co_ref[...] = _mm(c1h, Wc2[...]) + bc2[...]
        m_ref[...] = m

    return pl.pallas_call(
        body,
        grid=(grid,),
        in_specs=[
            _row_spec(BE, 128), _row_spec(BE, 128),
            _row_spec(BE, 16), _row_spec(BE, 16),
            _row_spec(BE, 64),
            _full_spec((16, 128)),
            _full_spec((1, 128)), _full_spec((64, 128)), _full_spec((1, 128)),
            _full_spec((128, 128)), _full_spec((1, 128)),
            _full_spec((128, 128)), _full_spec((1, 128)),
            _full_spec((128, 16)), _full_spec((1, 16)),
        ],
        out_specs=[_row_spec(BE, 128), _row_spec(BE, 16)],
        out_shape=[
            jax.ShapeDtypeStruct((ep, 128), F32),
            jax.ShapeDtypeStruct((ep, 16), F32),
        ],
    )


@functools.lru_cache(maxsize=None)
def _make_trans(ep):
    bp = BE // 8
    grid = ep // BE

    def body(xcgP, xrgP, coffP, P1B, P2B, OmB, SB0, SB1, SB2, t_ref):
        cd, cc, cv = _geometry_packed(
            xrgP[...], xcgP[...], P1B[...], P2B[...], OmB[...])
        cP = coffP[...]
        k0 = _mm(cP, SB0[...])
        k1 = _mm(cP, SB1[...])
        k2 = _mm(cP, SB2[...])
        t_ref[...] = cd * k0 + cc * k1 + cv * k2

    return pl.pallas_call(
        body,
        grid=(grid,),
        in_specs=[
            _row_spec(bp, 128), _row_spec(bp, 128), _row_spec(bp, 128),
            _full_spec((128, 128)), _full_spec((128, 128)),
            _full_spec((128, 128)),
            _full_spec((128, 128)), _full_spec((128, 128)),
            _full_spec((128, 128)),
        ],
        out_specs=[_row_spec(bp, 128)],
        out_shape=[jax.ShapeDtypeStruct((ep // 8, 128), F32)],
    )


@functools.lru_cache(maxsize=None)
def _make_centroid(rows, cols):
    def body(xt, xc_ref, cent_ref):
        v = xt[...]
        c = jnp.mean(v, axis=1, keepdims=True)
        cb = jnp.broadcast_to(c, v.shape)
        xc_ref[...] = v - cb
        cent_ref[...] = cb

    return pl.pallas_call(
        body,
        grid=(1,),
        in_specs=[_full_spec((rows, cols))],
        out_specs=[_full_spec((rows, cols)), _full_spec((rows, cols))],
        out_shape=[
            jax.ShapeDtypeStruct((rows, cols), F32),
            jax.ShapeDtypeStruct((rows, cols), F32),
        ],
    )


@functools.lru_cache(maxsize=None)
def _make_embed(n1):
    grid = n1 // BN

    def body(h, We, be, W1a, W1b, h1_ref, hA_ref, hB_ref):
        h1 = jnp.dot(h[...], We[...], preferred_element_type=F32) + be[...]
        h1_ref[...] = h1
        hA_ref[...] = _mm(h1, W1a[...])
        hB_ref[...] = _mm(h1, W1b[...])

    return pl.pallas_call(
        body,
        grid=(grid,),
        in_specs=[
            _row_spec(BN, 128),
            _full_spec((128, 128)), _full_spec((1, 128)),
            _full_spec((128, 128)), _full_spec((128, 128)),
        ],
        out_specs=[_row_spec(BN, 128)] * 3,
        out_shape=[
            jax.ShapeDtypeStruct((n1, 128), F32),
            jax.ShapeDtypeStruct((n1, 128), F32),
            jax.ShapeDtypeStruct((n1, 128), F32),
        ],
    )


def _node_update(h, a0, a1, Wn1a, Wn1b, bn1, Wn2, bn2, g, b):
    aggr = a0 + a1
    pre = (jnp.dot(h, Wn1a, preferred_element_type=F32)
           + jnp.dot(aggr, Wn1b, preferred_element_type=F32) + bn1)
    hn = jnp.dot(_silu(pre), Wn2, preferred_element_type=F32) + bn2
    mu = jnp.mean(hn, axis=1, keepdims=True)
    dev = hn - mu
    var = jnp.mean(dev * dev, axis=1, keepdims=True)
    return dev * jax.lax.rsqrt(var + 1e-5) * g + b


@functools.lru_cache(maxsize=None)
def _make_node(n1):
    grid = n1 // BN

    def body(h, a0, a1, d0, d1, xc, Wn1a, Wn1b, bn1, Wn2, bn2, g, b,
             W1a, W1b, h_ref, xc_ref, hA_ref, hB_ref):
        hnew = _node_update(h[...], a0[...], a1[...], Wn1a[...], Wn1b[...],
                            bn1[...], Wn2[...], bn2[...], g[...], b[...])
        h_ref[...] = hnew
        xc_ref[...] = xc[...] + d0[...] + d1[...]
        hA_ref[...] = _mm(hnew, W1a[...])
        hB_ref[...] = _mm(hnew, W1b[...])

    return pl.pallas_call(
        body,
        grid=(grid,),
        in_specs=[
            _row_spec(BN, 128),
            _row_spec(BN, 128), _row_spec(BN, 128),
            _row_spec(BN, 16), _row_spec(BN, 16), _row_spec(BN, 16),
            _full_spec((128, 128)), _full_spec((128, 128)), _full_spec((1, 128)),
            _full_spec((128, 128)), _full_spec((1, 128)),
            _full_spec((1, 128)), _full_spec((1, 128)),
            _full_spec((128, 128)), _full_spec((128, 128)),
        ],
        out_specs=[_row_spec(BN, 128), _row_spec(BN, 16),
                   _row_spec(BN, 128), _row_spec(BN, 128)],
        out_shape=[
            jax.ShapeDtypeStruct((n1, 128), F32),
            jax.ShapeDtypeStruct((n1, 16), F32),
            jax.ShapeDtypeStruct((n1, 128), F32),
            jax.ShapeDtypeStruct((n1, 128), F32),
        ],
    )


@functools.lru_cache(maxsize=None)
def _make_node_final(n1):
    grid = n1 // BN

    def body(h, a0, a1, d0, d1, xc, cent, Wn1a, Wn1b, bn1, Wn2, bn2, g, b,
             Wo, bo, ho_ref, xo_ref):
        hnew = _node_update(h[...], a0[...], a1[...], Wn1a[...], Wn1b[...],
                            bn1[...], Wn2[...], bn2[...], g[...], b[...])
        ho_ref[...] = jnp.dot(hnew, Wo[...], preferred_element_type=F32) + bo[...]
        xo_ref[...] = xc[...] + d0[...] + d1[...] + cent[...]

    return pl.pallas_call(
        body,
        grid=(grid,),
        in_specs=[
            _row_spec(BN, 128),
            _row_spec(BN, 128), _row_spec(BN, 128),
            _row_spec(BN, 16), _row_spec(BN, 16), _row_spec(BN, 16),
            _row_spec(BN, 16),
            _full_spec((128, 128)), _full_spec((128, 128)), _full_spec((1, 128)),
            _full_spec((128, 128)), _full_spec((1, 128)),
            _full_spec((1, 128)), _full_spec((1, 128)),
            _full_spec((128, 128)), _full_spec((1, 128)),
        ],
        out_specs=[_row_spec(BN, 128), _row_spec(BN, 16)],
        out_shape=[
            jax.ShapeDtypeStruct((n1, 128), F32),
            jax.ShapeDtypeStruct((n1, 16), F32),
        ],
    )


def _r2(v):
    return v.reshape(1, -1)


def kernel(h, edge_index, x, n_nodes, params):
    n, dmod = h.shape
    e = edge_index.shape[1]
    nn = 100
    nmol = n // nn

    n1 = ((n + 1 + NS * 8 - 1) // (NS * 8)) * (NS * 8)
    epm = NTILES * CH * 2
    ep = ((e + epm - 1) // epm) * epm

    row = edge_index[0].astype(jnp.int32)
    col = edge_index[1].astype(jnp.int32)
    pad_e = ep - e
    rowp = jnp.concatenate([row, jnp.full((pad_e,), n, jnp.int32)])
    colp = jnp.concatenate([col, jnp.full((pad_e,), n, jnp.int32)])

    xt = x.reshape(nmol, nn, 3).transpose(0, 2, 1).reshape(3 * nmol, nn)
    xc_rows, cent_rows = _make_centroid(3 * nmol, nn)(xt)
    xc0 = xc_rows.reshape(nmol, 3, nn).transpose(0, 2, 1).reshape(n, 3)
    cent = cent_rows.reshape(nmol, 3, nn).transpose(0, 2, 1).reshape(n, 3)
    zpadn = jnp.zeros((n1 - n, 3), F32)
    xcp = jnp.tile(jnp.concatenate([xc0, zpadn], 0), (1, 6))[:, :16]
    centp = jnp.pad(jnp.concatenate([cent, zpadn], 0), ((0, 0), (0, 13)))

    eye16 = jnp.eye(16, dtype=F32)
    eye8 = jnp.eye(8, dtype=F32)
    P1 = jnp.roll(eye16, 1, axis=0)
    P2 = jnp.roll(eye16, 2, axis=0)
    Om = jnp.concatenate([jnp.ones((3, 16), F32), jnp.zeros((13, 16), F32)], 0)
    Or = jnp.concatenate([jnp.ones((3, 128), F32), jnp.zeros((13, 128), F32)], 0)
    P1B = jnp.kron(eye8, P1)
    P2B = jnp.kron(eye8, P2)
    OmB = jnp.kron(eye8, Om)
    sels = [jnp.zeros((16, 16), F32).at[k].set(1.0) for k in range(3)]
    SB0, SB1, SB2 = (jnp.kron(eye8, s) for s in sels)
    Wf1 = params["fuse1"]["w"]
    MfP = jnp.concatenate(
        [jnp.kron(eye8, jnp.ones((16, 1), F32) @ (Wf1[j:j + 1] / 16.0))
         for j in range(8)], 0)
    Wf2P = jnp.kron(eye8, params["fuse2"]["w"])
    bf1P = jnp.tile(_r2(params["fuse1"]["b"]), (1, 8))
    bf2P = jnp.tile(_r2(params["fuse2"]["b"]), (1, 8))

    hpad = jnp.concatenate([h, jnp.zeros((n1 - n, dmod), F32)], 0)

    layers = params["layers"]

    def esplit(lp):
        W1 = lp["edge1"]["w"]
        return (_r2(W1[256]), W1[257:321],
                _r2(lp["edge1"]["b"]), lp["edge2"]["w"], _r2(lp["edge2"]["b"]),
                lp["coord1"]["w"], _r2(lp["coord1"]["b"]),
                jnp.pad(lp["coord2"]["w"], ((0, 0), (0, 13))),
                jnp.pad(_r2(lp["coord2"]["b"]), ((0, 0), (0, 13))))

    def nsplit(lp):
        Wn1 = lp["node1"]["w"]
        return (Wn1[0:128], Wn1[128:256], _r2(lp["node1"]["b"]),
                lp["node2"]["w"], _r2(lp["node2"]["b"]),
                _r2(lp["ln_g"]), _r2(lp["ln_b"]))

    W1a0, W1b0 = layers[0]["edge1"]["w"][0:128], layers[0]["edge1"]["w"][128:256]
    hcur, hA, hB = _make_embed(n1)(
        hpad, params["emb_in"]["w"], _r2(params["emb_in"]["b"]), W1a0, W1b0)

    zeros128 = jnp.zeros((n1, 128), F32)
    zeros16 = jnp.zeros((n1, 16), F32)

    gather = _make_gather(n1, ep)
    scatter = _make_scatter(n1, ep)
    ef0 = _make_ef0(ep)
    edgeK = _make_edge(ep)
    transK = _make_trans(ep)
    nodeK = _make_node(n1)
    nodeF = _make_node_final(n1)

    ef = None
    for li in range(4):
        lp = layers[li]
        sA, sB, xcg, xrg = gather(hA, hB, xcp, colp, rowp)
        xcgP = xcg.reshape(ep // 8, 128)
        xrgP = xrg.reshape(ep // 8, 128)
        ew = esplit(lp)
        if li == 0:
            (efP,) = ef0(xcgP, xrgP, P1B, P2B, OmB, MfP, bf1P, Wf2P, bf2P)
            ef = efP.reshape(ep, 64)
        m, coff = edgeK(sA, sB, xcg, xrg, ef, Or, *ew)
        coffP = coff.reshape(ep // 8, 128)
        (transP,) = transK(xcgP, xrgP, coffP, P1B, P2B, OmB, SB0, SB1, SB2)
        trans = transP.reshape(ep, 16)
        aggr2, delta2 = scatter(m, trans, colp, rowp, zeros128, zeros16)
        a0, a1 = aggr2[:n1], aggr2[n1:]
        d0, d1 = delta2[:n1], delta2[n1:]
        nw = nsplit(lp)
        if li < 3:
            Wn = layers[li + 1]["edge1"]["w"]
            hcur, xcp, hA, hB = nodeK(
                hcur, a0, a1, d0, d1, xcp, *nw, Wn[0:128], Wn[128:256])
        else:
            ho, xo16 = nodeF(
                hcur, a0, a1, d0, d1, xcp, centp, *nw,
                params["emb_out"]["w"], _r2(params["emb_out"]["b"]))

    return ho[:n], xo16[:n, :3]
